# baseline (device time: 144793 ns/iter reference)
import jax
import jax.numpy as jnp
from jax import lax
from jax.experimental import pallas as pl
from jax.experimental.pallas import tpu as pltpu

N_DEV = 8
B = 2
SQ = 128
D = 512
HL = 8
DH = 64
R = B * SQ


def kernel(x, Wq, Wo, K_ext, V_ext):
    my = lax.axis_index("i")
    K_loc = lax.dynamic_slice(K_ext, (0, 0, my * HL, 0), (B, SQ, HL, DH))
    V_loc = lax.dynamic_slice(V_ext, (0, 0, my * HL, 0), (B, SQ, HL, DH))
    K_loc = jnp.transpose(K_loc, (0, 2, 1, 3)).reshape(B * HL, SQ, DH)
    V_loc = jnp.transpose(V_loc, (0, 2, 1, 3)).reshape(B * HL, SQ, DH)
    x2d = x.reshape(R, D)

    def body(x_ref, wq_ref, wo_ref, k_ref, v_ref, out_ref,
             x_comm, acc_comm, o_scratch,
             xs_sems, xr_sems, as_sems, ar_sems, ack_sem):
        me = lax.axis_index("i")
        right = lax.rem(me + 1, N_DEV)
        left = lax.rem(me + N_DEV - 1, N_DEV)

        barrier_sem = pltpu.get_barrier_semaphore()
        for nbr in (left, right):
            pl.semaphore_signal(
                barrier_sem, inc=1,
                device_id=(nbr,), device_id_type=pl.DeviceIdType.MESH,
            )
        pl.semaphore_wait(barrier_sem, 2)

        def contrib(x_chunk):
            q_all = jnp.dot(x_chunk, wq_ref[...],
                            preferred_element_type=jnp.float32)
            for b in range(B):
                for h in range(HL):
                    q = q_all[b * SQ:(b + 1) * SQ, h * DH:(h + 1) * DH]
                    k = k_ref[b * HL + h]
                    v = v_ref[b * HL + h]
                    s = lax.dot_general(
                        q, k, (((1,), (1,)), ((), ())),
                        preferred_element_type=jnp.float32) * 0.125
                    m = jnp.max(s, axis=1, keepdims=True)
                    p = jnp.exp(s - m)
                    l = jnp.sum(p, axis=1, keepdims=True)
                    o = jnp.dot(p, v, preferred_element_type=jnp.float32) / l
                    o_scratch[b * SQ:(b + 1) * SQ, h * DH:(h + 1) * DH] = o
            return jnp.dot(o_scratch[...], wo_ref[...],
                           preferred_element_type=jnp.float32)

        x_comm[0] = x_ref[...]
        acc_comm[0] = contrib(x_ref[...])

        def hop(h, _):
            rdma_x = pltpu.make_async_remote_copy(
                src_ref=x_comm.at[h], dst_ref=x_comm.at[h + 1],
                send_sem=xs_sems.at[h], recv_sem=xr_sems.at[h],
                device_id=(right,), device_id_type=pl.DeviceIdType.MESH,
            )
            rdma_a = pltpu.make_async_remote_copy(
                src_ref=acc_comm.at[h], dst_ref=acc_comm.at[h + 1],
                send_sem=as_sems.at[h], recv_sem=ar_sems.at[h],
                device_id=(right,), device_id_type=pl.DeviceIdType.MESH,
            )
            rdma_x.start()
            rdma_a.start()
            rdma_x.wait()
            rdma_a.wait()
            c = contrib(x_comm.at[h + 1][...])
            acc_ref = acc_comm.at[h + 1]
            acc_ref[...] = acc_ref[...] + c
            return _

        lax.fori_loop(0, N_DEV - 1, hop, None)

        rdma_f = pltpu.make_async_remote_copy(
            src_ref=acc_comm.at[N_DEV - 1], dst_ref=out_ref,
            send_sem=as_sems.at[N_DEV - 1], recv_sem=ar_sems.at[N_DEV - 1],
            device_id=(right,), device_id_type=pl.DeviceIdType.MESH,
        )
        rdma_f.start()
        rdma_f.wait()

        pl.semaphore_signal(
            ack_sem, inc=1,
            device_id=(left,), device_id_type=pl.DeviceIdType.MESH,
        )
        pl.semaphore_wait(ack_sem, 1)

    out2d = pl.pallas_call(
        body,
        out_shape=jax.ShapeDtypeStruct((R, D), jnp.float32),
        in_specs=[pl.BlockSpec(memory_space=pltpu.VMEM)] * 5,
        out_specs=pl.BlockSpec(memory_space=pltpu.VMEM),
        scratch_shapes=[
            pltpu.VMEM((N_DEV, R, D), jnp.float32),
            pltpu.VMEM((N_DEV, R, D), jnp.float32),
            pltpu.VMEM((R, D), jnp.float32),
            pltpu.SemaphoreType.DMA((N_DEV,)),
            pltpu.SemaphoreType.DMA((N_DEV,)),
            pltpu.SemaphoreType.DMA((N_DEV,)),
            pltpu.SemaphoreType.DMA((N_DEV,)),
            pltpu.SemaphoreType.REGULAR,
        ],
        compiler_params=pltpu.CompilerParams(collective_id=0),
    )(x2d, Wq, Wo, K_loc, V_loc)

    return out2d.reshape(B, SQ, D)


# device time: 58966 ns/iter; 2.4555x vs baseline; 2.4555x over previous
import jax
import jax.numpy as jnp
from jax import lax
from jax.experimental import pallas as pl
from jax.experimental.pallas import tpu as pltpu

N_DEV = 8
B = 2
SQ = 128
D = 512
HL = 8
DH = 64


def kernel(x, Wq, Wo, K_ext, V_ext):
    my = lax.axis_index("i")
    K_loc = lax.dynamic_slice(K_ext, (0, 0, my * HL, 0), (B, SQ, HL, DH))
    V_loc = lax.dynamic_slice(V_ext, (0, 0, my * HL, 0), (B, SQ, HL, DH))
    K_loc = jnp.transpose(K_loc, (0, 2, 1, 3)).reshape(B * HL, SQ, DH)
    V_loc = jnp.transpose(V_loc, (0, 2, 1, 3)).reshape(B * HL, SQ, DH)

    def body(x_ref, wq_ref, wo_ref, k_ref, v_ref, out_ref,
             x_cR, x_cL, a_cR, a_cL, o_scr,
             xRs, xRr, aRs, aRr, xLs, xLr, aLs, aLr):
        me = lax.axis_index("i")
        right = lax.rem(me + 1, N_DEV)
        left = lax.rem(me + N_DEV - 1, N_DEV)

        barrier_sem = pltpu.get_barrier_semaphore()
        for nbr in (left, right):
            pl.semaphore_signal(
                barrier_sem, inc=1,
                device_id=(nbr,), device_id_type=pl.DeviceIdType.MESH,
            )
        pl.semaphore_wait(barrier_sem, 2)

        def marc(comm, i, send_sems, recv_sems, dev):
            return pltpu.make_async_remote_copy(
                src_ref=comm.at[i], dst_ref=comm.at[i + 1],
                send_sem=send_sems.at[i], recv_sem=recv_sems.at[i],
                device_id=(dev,), device_id_type=pl.DeviceIdType.MESH,
            )

        def m_xR(i):
            return marc(x_cR, i, xRs, xRr, right)

        def m_aR(i):
            return marc(a_cR, i, aRs, aRr, right)

        def m_xL(i):
            return marc(x_cL, i, xLs, xLr, left)

        def m_aL(i):
            return marc(a_cL, i, aLs, aLr, left)

        def contrib(x_b, b):
            q_all = jnp.dot(x_b, wq_ref[...],
                            preferred_element_type=jnp.float32)
            for h in range(HL):
                q = q_all[:, h * DH:(h + 1) * DH]
                k = k_ref[b * HL + h]
                v = v_ref[b * HL + h]
                s = lax.dot_general(
                    q, k, (((1,), (1,)), ((), ())),
                    preferred_element_type=jnp.float32) * 0.125
                m = jnp.max(s, axis=1, keepdims=True)
                p = jnp.exp(s - m)
                l = jnp.sum(p, axis=1, keepdims=True)
                o = jnp.dot(p, v, preferred_element_type=jnp.float32) / l
                o_scr[:, h * DH:(h + 1) * DH] = o
            return jnp.dot(o_scr[...], wo_ref[...],
                           preferred_element_type=jnp.float32)

        x_cR[0] = x_ref[0]
        x_cL[0] = x_ref[1]
        m_xR(0).start()
        m_xL(0).start()
        a_cR[0] = contrib(x_ref[0], 0)
        m_aR(0).start()
        a_cL[0] = contrib(x_ref[1], 1)
        m_aL(0).start()

        def round_(h, _):
            m_xR(h - 1).wait_recv()
            m_xR(h).start()
            m_xL(h - 1).wait_recv()
            m_xL(h).start()
            cR = contrib(x_cR.at[h][...], 0)
            m_aR(h - 1).wait_recv()
            aref = a_cR.at[h]
            aref[...] = aref[...] + cR
            m_aR(h).start()
            cL = contrib(x_cL.at[h][...], 1)
            m_aL(h - 1).wait_recv()
            aref = a_cL.at[h]
            aref[...] = aref[...] + cL
            m_aL(h).start()
            return _

        lax.fori_loop(1, N_DEV - 1, round_, None)

        m_xR(6).wait_recv()
        cR = contrib(x_cR.at[7][...], 0)
        m_aR(6).wait_recv()
        aref = a_cR.at[7]
        aref[...] = aref[...] + cR
        finR = pltpu.make_async_remote_copy(
            src_ref=a_cR.at[7], dst_ref=out_ref.at[0],
            send_sem=aRs.at[7], recv_sem=aRr.at[7],
            device_id=(right,), device_id_type=pl.DeviceIdType.MESH,
        )
        finR.start()

        m_xL(6).wait_recv()
        cL = contrib(x_cL.at[7][...], 1)
        m_aL(6).wait_recv()
        aref = a_cL.at[7]
        aref[...] = aref[...] + cL
        finL = pltpu.make_async_remote_copy(
            src_ref=a_cL.at[7], dst_ref=out_ref.at[1],
            send_sem=aLs.at[7], recv_sem=aLr.at[7],
            device_id=(left,), device_id_type=pl.DeviceIdType.MESH,
        )
        finL.start()

        finR.wait_recv()
        finL.wait_recv()

        def drain(h, _):
            m_xR(h).wait_send()
            m_xL(h).wait_send()
            m_aR(h).wait_send()
            m_aL(h).wait_send()
            return _

        lax.fori_loop(0, N_DEV - 1, drain, None)
        finR.wait_send()
        finL.wait_send()

    out = pl.pallas_call(
        body,
        out_shape=jax.ShapeDtypeStruct((B, SQ, D), jnp.float32),
        in_specs=[pl.BlockSpec(memory_space=pltpu.VMEM)] * 5,
        out_specs=pl.BlockSpec(memory_space=pltpu.VMEM),
        scratch_shapes=[
            pltpu.VMEM((N_DEV, SQ, D), jnp.float32),
            pltpu.VMEM((N_DEV, SQ, D), jnp.float32),
            pltpu.VMEM((N_DEV, SQ, D), jnp.float32),
            pltpu.VMEM((N_DEV, SQ, D), jnp.float32),
            pltpu.VMEM((SQ, D), jnp.float32),
            pltpu.SemaphoreType.DMA((N_DEV,)),
            pltpu.SemaphoreType.DMA((N_DEV,)),
            pltpu.SemaphoreType.DMA((N_DEV,)),
            pltpu.SemaphoreType.DMA((N_DEV,)),
            pltpu.SemaphoreType.DMA((N_DEV,)),
            pltpu.SemaphoreType.DMA((N_DEV,)),
            pltpu.SemaphoreType.DMA((N_DEV,)),
            pltpu.SemaphoreType.DMA((N_DEV,)),
        ],
        compiler_params=pltpu.CompilerParams(collective_id=0),
    )(x, Wq, Wo, K_loc, V_loc)

    return out


# device time: 51658 ns/iter; 2.8029x vs baseline; 1.1415x over previous
import jax
import jax.numpy as jnp
from jax import lax
from jax.experimental import pallas as pl
from jax.experimental.pallas import tpu as pltpu

N_DEV = 8
B = 2
SQ = 128
D = 512
HL = 8
DH = 64


def kernel(x, Wq, Wo, K_ext, V_ext):
    my = lax.axis_index("i")
    K_loc = lax.dynamic_slice(K_ext, (0, 0, my * HL, 0), (B, SQ, HL, DH))
    V_loc = lax.dynamic_slice(V_ext, (0, 0, my * HL, 0), (B, SQ, HL, DH))
    K_loc = jnp.transpose(K_loc, (0, 2, 1, 3)).reshape(B * HL, SQ, DH)
    V_loc = jnp.transpose(V_loc, (0, 2, 1, 3)).reshape(B * HL, SQ, DH)
    bf = jnp.bfloat16
    x = x.astype(bf)
    Wq = Wq.astype(bf)
    Wo = Wo.astype(bf)
    K_loc = K_loc.astype(bf)
    V_loc = V_loc.astype(bf)

    def body(x_ref, wq_ref, wo_ref, k_ref, v_ref, out_ref,
             x_cR, x_cL, a_cR, a_cL, o_scr,
             xRs, xRr, aRs, aRr, xLs, xLr, aLs, aLr):
        me = lax.axis_index("i")
        right = lax.rem(me + 1, N_DEV)
        left = lax.rem(me + N_DEV - 1, N_DEV)

        barrier_sem = pltpu.get_barrier_semaphore()
        for nbr in (left, right):
            pl.semaphore_signal(
                barrier_sem, inc=1,
                device_id=(nbr,), device_id_type=pl.DeviceIdType.MESH,
            )
        pl.semaphore_wait(barrier_sem, 2)

        def marc(comm, i, send_sems, recv_sems, dev):
            return pltpu.make_async_remote_copy(
                src_ref=comm.at[i], dst_ref=comm.at[i + 1],
                send_sem=send_sems.at[i], recv_sem=recv_sems.at[i],
                device_id=(dev,), device_id_type=pl.DeviceIdType.MESH,
            )

        def m_xR(i):
            return marc(x_cR, i, xRs, xRr, right)

        def m_aR(i):
            return marc(a_cR, i, aRs, aRr, right)

        def m_xL(i):
            return marc(x_cL, i, xLs, xLr, left)

        def m_aL(i):
            return marc(a_cL, i, aLs, aLr, left)

        def contrib(x_b, b):
            q_all = jnp.dot(x_b, wq_ref[...],
                            preferred_element_type=jnp.float32
                            ).astype(jnp.bfloat16)
            for h in range(HL):
                q = q_all[:, h * DH:(h + 1) * DH]
                k = k_ref[b * HL + h]
                v = v_ref[b * HL + h]
                s = lax.dot_general(
                    q, k, (((1,), (1,)), ((), ())),
                    preferred_element_type=jnp.float32) * 0.125
                m = jnp.max(s, axis=1, keepdims=True)
                p = jnp.exp(s - m)
                l = jnp.sum(p, axis=1, keepdims=True)
                o = jnp.dot(p.astype(jnp.bfloat16), v,
                            preferred_element_type=jnp.float32) / l
                o_scr[:, h * DH:(h + 1) * DH] = o.astype(jnp.bfloat16)
            return jnp.dot(o_scr[...], wo_ref[...],
                           preferred_element_type=jnp.float32)

        x_cR[0] = x_ref[0]
        x_cL[0] = x_ref[1]
        m_xR(0).start()
        m_xL(0).start()
        a_cR[0] = contrib(x_ref[0], 0).astype(jnp.bfloat16)
        m_aR(0).start()
        a_cL[0] = contrib(x_ref[1], 1).astype(jnp.bfloat16)
        m_aL(0).start()

        def round_(h, _):
            m_xR(h - 1).wait_recv()
            m_xR(h).start()
            m_xL(h - 1).wait_recv()
            m_xL(h).start()
            cR = contrib(x_cR.at[h][...], 0)
            m_aR(h - 1).wait_recv()
            aref = a_cR.at[h]
            aref[...] = (aref[...].astype(jnp.float32) + cR
                         ).astype(jnp.bfloat16)
            m_aR(h).start()
            cL = contrib(x_cL.at[h][...], 1)
            m_aL(h - 1).wait_recv()
            aref = a_cL.at[h]
            aref[...] = (aref[...].astype(jnp.float32) + cL
                         ).astype(jnp.bfloat16)
            m_aL(h).start()
            return _

        lax.fori_loop(1, N_DEV - 1, round_, None)

        m_xR(6).wait_recv()
        cR = contrib(x_cR.at[7][...], 0)
        m_aR(6).wait_recv()
        aref = a_cR.at[7]
        aref[...] = (aref[...].astype(jnp.float32) + cR).astype(jnp.bfloat16)
        finR = pltpu.make_async_remote_copy(
            src_ref=a_cR.at[7], dst_ref=out_ref.at[0],
            send_sem=aRs.at[7], recv_sem=aRr.at[7],
            device_id=(right,), device_id_type=pl.DeviceIdType.MESH,
        )
        finR.start()

        m_xL(6).wait_recv()
        cL = contrib(x_cL.at[7][...], 1)
        m_aL(6).wait_recv()
        aref = a_cL.at[7]
        aref[...] = (aref[...].astype(jnp.float32) + cL).astype(jnp.bfloat16)
        finL = pltpu.make_async_remote_copy(
            src_ref=a_cL.at[7], dst_ref=out_ref.at[1],
            send_sem=aLs.at[7], recv_sem=aLr.at[7],
            device_id=(left,), device_id_type=pl.DeviceIdType.MESH,
        )
        finL.start()

        finR.wait_recv()
        finL.wait_recv()

        def drain(h, _):
            m_xR(h).wait_send()
            m_xL(h).wait_send()
            m_aR(h).wait_send()
            m_aL(h).wait_send()
            return _

        lax.fori_loop(0, N_DEV - 1, drain, None)
        finR.wait_send()
        finL.wait_send()

    out = pl.pallas_call(
        body,
        out_shape=jax.ShapeDtypeStruct((B, SQ, D), jnp.bfloat16),
        in_specs=[pl.BlockSpec(memory_space=pltpu.VMEM)] * 5,
        out_specs=pl.BlockSpec(memory_space=pltpu.VMEM),
        scratch_shapes=[
            pltpu.VMEM((N_DEV, SQ, D), jnp.bfloat16),
            pltpu.VMEM((N_DEV, SQ, D), jnp.bfloat16),
            pltpu.VMEM((N_DEV, SQ, D), jnp.bfloat16),
            pltpu.VMEM((N_DEV, SQ, D), jnp.bfloat16),
            pltpu.VMEM((SQ, D), jnp.bfloat16),
            pltpu.SemaphoreType.DMA((N_DEV,)),
            pltpu.SemaphoreType.DMA((N_DEV,)),
            pltpu.SemaphoreType.DMA((N_DEV,)),
            pltpu.SemaphoreType.DMA((N_DEV,)),
            pltpu.SemaphoreType.DMA((N_DEV,)),
            pltpu.SemaphoreType.DMA((N_DEV,)),
            pltpu.SemaphoreType.DMA((N_DEV,)),
            pltpu.SemaphoreType.DMA((N_DEV,)),
        ],
        compiler_params=pltpu.CompilerParams(collective_id=0),
    )(x, Wq, Wo, K_loc, V_loc)

    return out.astype(jnp.float32)


# device time: 43060 ns/iter; 3.3626x vs baseline; 1.1997x over previous
import jax
import jax.numpy as jnp
from jax import lax
from jax.experimental import pallas as pl
from jax.experimental.pallas import tpu as pltpu

N_DEV = 8
B = 2
SQ = 128
D = 512
HL = 8
DH = 64


def kernel(x, Wq, Wo, K_ext, V_ext):
    my = lax.axis_index("i")
    K_loc = lax.dynamic_slice(K_ext, (0, 0, my * HL, 0), (B, SQ, HL, DH))
    V_loc = lax.dynamic_slice(V_ext, (0, 0, my * HL, 0), (B, SQ, HL, DH))
    K_loc = jnp.transpose(K_loc, (0, 2, 1, 3)).reshape(B * HL, SQ, DH)
    V_loc = jnp.transpose(V_loc, (0, 2, 1, 3)).reshape(B * HL, SQ, DH)
    bf = jnp.bfloat16
    x = x.astype(bf)
    Wq = Wq.astype(bf)
    Wo = Wo.astype(bf)
    K_loc = K_loc.astype(bf)
    V_loc = V_loc.astype(bf)

    def body(x_ref, wq_ref, wo_ref, k_ref, v_ref, out_ref,
             x_cR, x_cL, a_cR, a_cL, q_in, o_scr,
             xRs, xRr, aRs, aRr, xLs, xLr, aLs, aLr):
        me = lax.axis_index("i")
        right = lax.rem(me + 1, N_DEV)
        left = lax.rem(me + N_DEV - 1, N_DEV)

        barrier_sem = pltpu.get_barrier_semaphore()
        for nbr in (left, right):
            pl.semaphore_signal(
                barrier_sem, inc=1,
                device_id=(nbr,), device_id_type=pl.DeviceIdType.MESH,
            )
        pl.semaphore_wait(barrier_sem, 2)

        def marc(comm, i, send_sems, recv_sems, dev):
            return pltpu.make_async_remote_copy(
                src_ref=comm.at[i], dst_ref=comm.at[i + 1],
                send_sem=send_sems.at[i], recv_sem=recv_sems.at[i],
                device_id=(dev,), device_id_type=pl.DeviceIdType.MESH,
            )

        def m_xR(i):
            return marc(x_cR, i, xRs, xRr, right)

        def m_aR(i):
            return marc(a_cR, i, aRs, aRr, right)

        def m_xL(i):
            return marc(x_cL, i, xLs, xLr, left)

        def m_aL(i):
            return marc(a_cL, i, aLs, aLr, left)

        def contrib_pair(xR, xL):
            q_in[0:SQ, :] = xR
            q_in[SQ:2 * SQ, :] = xL
            q = jnp.dot(q_in[...], wq_ref[...],
                        preferred_element_type=jnp.float32
                        ).astype(jnp.bfloat16)
            q4 = q.reshape(B, SQ, HL, DH)
            k4 = k_ref[...].reshape(B, HL, SQ, DH)
            v4 = v_ref[...].reshape(B, HL, SQ, DH)
            for h in range(HL):
                qh = q4[:, :, h, :]
                s = lax.dot_general(
                    qh, k4[:, h], (((2,), (2,)), ((0,), (0,))),
                    preferred_element_type=jnp.float32) * 0.125
                m = jnp.max(s, axis=2, keepdims=True)
                p = jnp.exp(s - m)
                l = jnp.sum(p, axis=2, keepdims=True)
                o = lax.dot_general(
                    p.astype(jnp.bfloat16), v4[:, h],
                    (((2,), (1,)), ((0,), (0,))),
                    preferred_element_type=jnp.float32) / l
                o_scr[:, :, h * DH:(h + 1) * DH] = o.astype(jnp.bfloat16)
            o2 = o_scr[...].reshape(B * SQ, HL * DH)
            return jnp.dot(o2, wo_ref[...],
                           preferred_element_type=jnp.float32
                           ).reshape(B, SQ, D)

        x_cR[0] = x_ref[0]
        x_cL[0] = x_ref[1]
        m_xR(0).start()
        m_xL(0).start()
        c = contrib_pair(x_ref[0], x_ref[1])
        a_cR[0] = c[0].astype(jnp.bfloat16)
        m_aR(0).start()
        a_cL[0] = c[1].astype(jnp.bfloat16)
        m_aL(0).start()

        def round_(h, _):
            m_xR(h - 1).wait_recv()
            m_xR(h).start()
            m_xL(h - 1).wait_recv()
            m_xL(h).start()
            c = contrib_pair(x_cR.at[h][...], x_cL.at[h][...])
            m_aR(h - 1).wait_recv()
            aref = a_cR.at[h]
            aref[...] = (aref[...].astype(jnp.float32) + c[0]
                         ).astype(jnp.bfloat16)
            m_aR(h).start()
            m_aL(h - 1).wait_recv()
            aref = a_cL.at[h]
            aref[...] = (aref[...].astype(jnp.float32) + c[1]
                         ).astype(jnp.bfloat16)
            m_aL(h).start()
            return _

        lax.fori_loop(1, N_DEV - 1, round_, None)

        m_xR(6).wait_recv()
        m_xL(6).wait_recv()
        c = contrib_pair(x_cR.at[7][...], x_cL.at[7][...])
        m_aR(6).wait_recv()
        aref = a_cR.at[7]
        aref[...] = (aref[...].astype(jnp.float32) + c[0]).astype(jnp.bfloat16)
        finR = pltpu.make_async_remote_copy(
            src_ref=a_cR.at[7], dst_ref=out_ref.at[0],
            send_sem=aRs.at[7], recv_sem=aRr.at[7],
            device_id=(right,), device_id_type=pl.DeviceIdType.MESH,
        )
        finR.start()

        m_aL(6).wait_recv()
        aref = a_cL.at[7]
        aref[...] = (aref[...].astype(jnp.float32) + c[1]).astype(jnp.bfloat16)
        finL = pltpu.make_async_remote_copy(
            src_ref=a_cL.at[7], dst_ref=out_ref.at[1],
            send_sem=aLs.at[7], recv_sem=aLr.at[7],
            device_id=(left,), device_id_type=pl.DeviceIdType.MESH,
        )
        finL.start()

        finR.wait_recv()
        finL.wait_recv()

        def drain(h, _):
            m_xR(h).wait_send()
            m_xL(h).wait_send()
            m_aR(h).wait_send()
            m_aL(h).wait_send()
            return _

        lax.fori_loop(0, N_DEV - 1, drain, None)
        finR.wait_send()
        finL.wait_send()

    out = pl.pallas_call(
        body,
        out_shape=jax.ShapeDtypeStruct((B, SQ, D), jnp.bfloat16),
        in_specs=[pl.BlockSpec(memory_space=pltpu.VMEM)] * 5,
        out_specs=pl.BlockSpec(memory_space=pltpu.VMEM),
        scratch_shapes=[
            pltpu.VMEM((N_DEV, SQ, D), jnp.bfloat16),
            pltpu.VMEM((N_DEV, SQ, D), jnp.bfloat16),
            pltpu.VMEM((N_DEV, SQ, D), jnp.bfloat16),
            pltpu.VMEM((N_DEV, SQ, D), jnp.bfloat16),
            pltpu.VMEM((B * SQ, D), jnp.bfloat16),
            pltpu.VMEM((B, SQ, HL * DH), jnp.bfloat16),
            pltpu.SemaphoreType.DMA((N_DEV,)),
            pltpu.SemaphoreType.DMA((N_DEV,)),
            pltpu.SemaphoreType.DMA((N_DEV,)),
            pltpu.SemaphoreType.DMA((N_DEV,)),
            pltpu.SemaphoreType.DMA((N_DEV,)),
            pltpu.SemaphoreType.DMA((N_DEV,)),
            pltpu.SemaphoreType.DMA((N_DEV,)),
            pltpu.SemaphoreType.DMA((N_DEV,)),
        ],
        compiler_params=pltpu.CompilerParams(collective_id=0),
    )(x, Wq, Wo, K_loc, V_loc)

    return out.astype(jnp.float32)
